# Initial kernel scaffold; baseline (speedup 1.0000x reference)
import jax
import jax.numpy as jnp
from jax.experimental import pallas as pl


def kernel(x, edge_features, E_idx, W1, b1, W2, b2):
    raise NotImplementedError("write your pallas kernel here")


# speedup vs baseline: 12.0125x; 12.0125x over previous
import functools

import jax
import jax.numpy as jnp
from jax import lax
from jax.experimental import pallas as pl
from jax.experimental.pallas import tpu as pltpu
from jax.experimental.pallas import tpu_sc as plsc

_NC = 2
_NS = 16
_NW = _NC * _NS
_CH = 128
_STAGES = 4


def _sc_gather(table, idx3, R, C, dtype):
    nchunk = idx3.shape[1]
    rows_per_w = nchunk * _CH
    mesh = plsc.VectorSubcoreMesh(
        core_axis_name="c", subcore_axis_name="s", num_cores=_NC, num_subcores=_NS
    )

    @functools.partial(
        pl.kernel,
        mesh=mesh,
        out_type=jax.ShapeDtypeStruct((R, C), dtype),
        scratch_types=[
            pltpu.VMEM((nchunk, _CH), jnp.int32),
            pltpu.VMEM((_CH, C), dtype),
            pltpu.VMEM((_CH, C), dtype),
            pltpu.SemaphoreType.DMA,
            pltpu.SemaphoreType.DMA,
        ],
    )
    def k(table_hbm, idx_hbm, out_hbm, idx_v, rows0, rows1, sem0, sem1):
        wid = lax.axis_index("s") * _NC + lax.axis_index("c")
        base = wid * rows_per_w
        pltpu.sync_copy(idx_hbm.at[wid], idx_v)
        bufs = (rows0, rows1)
        sems = (sem0, sem1)
        dummy_src = table_hbm.at[pl.ds(0, _CH)]
        pltpu.make_async_copy(table_hbm.at[idx_v.at[0]], rows0, sem0).start()

        def process(jj, b):
            pltpu.make_async_copy(
                table_hbm.at[idx_v.at[jj + 1]], bufs[1 - b], sems[1 - b]
            ).start()
            pltpu.make_async_copy(dummy_src, bufs[b], sems[b]).wait()
            pltpu.sync_copy(bufs[b], out_hbm.at[pl.ds(base + jj * _CH, _CH)])

        def body(j, _):
            process(j * 2, 0)
            process(j * 2 + 1, 1)
            return 0

        lax.fori_loop(0, nchunk // 2 - 1, body, 0, unroll=False)
        process(nchunk - 2, 0)
        pltpu.make_async_copy(dummy_src, bufs[1], sems[1]).wait()
        pltpu.sync_copy(
            bufs[1], out_hbm.at[pl.ds(base + (nchunk - 1) * _CH, _CH)]
        )

    return k(table, idx3)


def _gelu_exact(v):
    return 0.5 * v * (1.0 + lax.erf(v * 0.7071067811865476))


def _tc_body(K, ef_ref, xj_ref, w1_ref, b1_ref, w2_ref, b2_ref, out_ref):
    w1 = w1_ref[...]
    w2 = w2_ref[...]
    b1 = b1_ref[...]
    b2 = b2_ref[...]
    nb = out_ref.shape[0]
    C = out_ref.shape[1]
    e = ef_ref[...]
    h = _gelu_exact(jnp.dot(e, w1, preferred_element_type=jnp.float32) + b1)
    f = _gelu_exact(jnp.dot(h, w2, preferred_element_type=jnp.float32) + b2)
    prod = f * xj_ref[...]
    out_ref[...] = jnp.sum(prod.reshape(nb, K, C), axis=1)


def kernel(x, edge_features, E_idx, W1, b1, W2, b2):
    B, N, C = x.shape
    K = E_idx.shape[-1]
    R = B * N * K

    table = x.reshape(B * N, C)
    offs = (jnp.arange(B, dtype=jnp.int32) * N)[:, None, None]
    idx_flat = (E_idx + offs).reshape(R)

    nodes_s = B * N // _STAGES
    rows_s = nodes_s * K
    nchunk = rows_s // (_NW * _CH)
    ef2 = edge_features.reshape(B * N * K, C)

    nb = 256
    tc = pl.pallas_call(
        functools.partial(_tc_body, K),
        grid=(nodes_s // nb,),
        in_specs=[
            pl.BlockSpec((nb * K, C), lambda i: (i, 0)),
            pl.BlockSpec((nb * K, C), lambda i: (i, 0)),
            pl.BlockSpec((C, C), lambda i: (0, 0)),
            pl.BlockSpec((1, C), lambda i: (0, 0)),
            pl.BlockSpec((C, C), lambda i: (0, 0)),
            pl.BlockSpec((1, C), lambda i: (0, 0)),
        ],
        out_specs=pl.BlockSpec((nb, C), lambda i: (i, 0)),
        out_shape=jax.ShapeDtypeStruct((nodes_s, C), jnp.float32),
    )

    w1t = W1.T
    w2t = W2.T
    b1r = b1.reshape(1, C)
    b2r = b2.reshape(1, C)
    outs = []
    for s in range(_STAGES):
        idx3 = lax.dynamic_slice_in_dim(idx_flat, s * rows_s, rows_s).reshape(
            _NW, nchunk, _CH
        )
        xj = _sc_gather(table, idx3, rows_s, C, jnp.float32)
        ef_s = lax.dynamic_slice_in_dim(ef2, s * rows_s, rows_s)
        outs.append(tc(ef_s, xj, w1t, b1r, w2t, b2r))
    return jnp.concatenate(outs, axis=0).reshape(B, N, C)


# speedup vs baseline: 16.4741x; 1.3714x over previous
import functools

import jax
import jax.numpy as jnp
from jax import lax
from jax.experimental import pallas as pl
from jax.experimental.pallas import tpu as pltpu
from jax.experimental.pallas import tpu_sc as plsc

_NC = 2
_NS = 16
_NW = _NC * _NS
_CH = 128
_STAGES = 4


def _sc_gather(table, idx3, R, C, dtype):
    nchunk = idx3.shape[1]
    rows_per_w = nchunk * _CH
    mesh = plsc.VectorSubcoreMesh(
        core_axis_name="c", subcore_axis_name="s", num_cores=_NC, num_subcores=_NS
    )

    @functools.partial(
        pl.kernel,
        mesh=mesh,
        out_type=jax.ShapeDtypeStruct((R, C), dtype),
        scratch_types=[
            pltpu.VMEM((nchunk, _CH), jnp.int32),
            pltpu.VMEM((_CH, C), dtype),
            pltpu.VMEM((_CH, C), dtype),
            pltpu.SemaphoreType.DMA,
            pltpu.SemaphoreType.DMA,
        ],
    )
    def k(table_hbm, idx_hbm, out_hbm, idx_v, rows0, rows1, sem0, sem1):
        wid = lax.axis_index("s") * _NC + lax.axis_index("c")
        base = wid * rows_per_w
        pltpu.sync_copy(idx_hbm.at[wid], idx_v)
        bufs = (rows0, rows1)
        sems = (sem0, sem1)
        dummy_src = table_hbm.at[pl.ds(0, _CH)]
        pltpu.make_async_copy(table_hbm.at[idx_v.at[0]], rows0, sem0).start()

        def process(jj, b):
            pltpu.make_async_copy(
                table_hbm.at[idx_v.at[jj + 1]], bufs[1 - b], sems[1 - b]
            ).start()
            pltpu.make_async_copy(dummy_src, bufs[b], sems[b]).wait()
            pltpu.sync_copy(bufs[b], out_hbm.at[pl.ds(base + jj * _CH, _CH)])

        def body(j, _):
            process(j * 2, 0)
            process(j * 2 + 1, 1)
            return 0

        lax.fori_loop(0, nchunk // 2 - 1, body, 0, unroll=False)
        process(nchunk - 2, 0)
        pltpu.make_async_copy(dummy_src, bufs[1], sems[1]).wait()
        pltpu.sync_copy(
            bufs[1], out_hbm.at[pl.ds(base + (nchunk - 1) * _CH, _CH)]
        )

    return k(table, idx3)


def _gelu_exact(v):
    return 0.5 * v * (1.0 + lax.erf(v * 0.7071067811865476))


def _tc_body(K, ef_ref, xj_ref, w1_ref, b1_ref, w2_ref, b2_ref, out_ref):
    w1 = w1_ref[...]
    w2 = w2_ref[...]
    b1 = b1_ref[...]
    b2 = b2_ref[...]
    nb = out_ref.shape[0]
    C = out_ref.shape[1]
    e = ef_ref[...]
    h = _gelu_exact(jnp.dot(e, w1, preferred_element_type=jnp.float32) + b1)
    f = _gelu_exact(jnp.dot(h, w2, preferred_element_type=jnp.float32) + b2)
    prod = f * xj_ref[...]
    out_ref[...] = jnp.sum(prod.reshape(nb, K, C), axis=1)


def kernel(x, edge_features, E_idx, W1, b1, W2, b2):
    B, N, C = x.shape
    K = E_idx.shape[-1]
    R = B * N * K

    table = x.reshape(B * N, C)
    offs = (jnp.arange(B, dtype=jnp.int32) * N)[:, None, None]
    idx_flat = (E_idx + offs).reshape(R)

    nodes_s = B * N // _STAGES
    rows_s = nodes_s * K
    nchunk = rows_s // (_NW * _CH)
    ef2 = edge_features.reshape(B * N * K, C)

    nb = 256
    blocks_s = nodes_s // nb

    w1t = W1.T
    w2t = W2.T
    b1r = b1.reshape(1, C)
    b2r = b2.reshape(1, C)
    idx4 = idx_flat.reshape(_STAGES, _NW, nchunk, _CH)
    outs = []
    for s in range(_STAGES):
        xj = _sc_gather(table, idx4[s], rows_s, C, jnp.float32)
        tc = pl.pallas_call(
            functools.partial(_tc_body, K),
            grid=(blocks_s,),
            in_specs=[
                pl.BlockSpec((nb * K, C), lambda i, s=s: (s * blocks_s + i, 0)),
                pl.BlockSpec((nb * K, C), lambda i: (i, 0)),
                pl.BlockSpec((C, C), lambda i: (0, 0)),
                pl.BlockSpec((1, C), lambda i: (0, 0)),
                pl.BlockSpec((C, C), lambda i: (0, 0)),
                pl.BlockSpec((1, C), lambda i: (0, 0)),
            ],
            out_specs=pl.BlockSpec((nb, C), lambda i: (i, 0)),
            out_shape=jax.ShapeDtypeStruct((nodes_s, C), jnp.float32),
        )
        outs.append(tc(ef2, xj, w1t, b1r, w2t, b2r))
    return jnp.concatenate(outs, axis=0).reshape(B, N, C)
